# Initial kernel scaffold; baseline (speedup 1.0000x reference)
import jax
import jax.numpy as jnp
from jax.experimental import pallas as pl


def kernel(x, y):
    raise NotImplementedError("write your pallas kernel here")


# speedup vs baseline: 1.6565x; 1.6565x over previous
import functools

import jax
import jax.numpy as jnp
from jax import lax
from jax.experimental import pallas as pl
from jax.experimental.pallas import tpu as pltpu
from jax.experimental.pallas import tpu_sc as plsc

_T = 500000
_H = 128
_NCLS = 10
_THR = 4.0
_CLIP = 100.0

_BR = 4000
_NBLK = _T // _BR


def _final_stats(s8, q8, n):
    s = jnp.sum(s8, axis=0, keepdims=True)
    q = jnp.sum(q8, axis=0, keepdims=True)
    m = s / n
    v = jnp.maximum((q - n * m * m) / (n - 1.0), 0.0)
    sd = jnp.maximum(jnp.sqrt(v), 1e-6)
    return m, sd


def _masked_bounds(ms8, mq8, mc8):
    s = jnp.sum(ms8, axis=0, keepdims=True)
    q = jnp.sum(mq8, axis=0, keepdims=True)
    c = jnp.sum(mc8, axis=0, keepdims=True)
    m = s / c
    v = jnp.maximum((q - c * m * m) / (c - 1.0), 0.0)
    sd = jnp.maximum(jnp.sqrt(v), 1e-6)
    return m - _THR * sd, m + _THR * sd


def _p1_body(x_ref, s_ref, q_ref):
    @pl.when(pl.program_id(0) == 0)
    def _():
        s_ref[...] = jnp.zeros_like(s_ref)
        q_ref[...] = jnp.zeros_like(q_ref)

    x3 = x_ref[...].reshape(_BR // 8, 8, _H)
    s_ref[...] += jnp.sum(x3, axis=0)
    q_ref[...] += jnp.sum(x3 * x3, axis=0)


def _p2_body(x_ref, s_ref, q_ref, ms_ref, mq_ref, mc_ref):
    m, sd = _final_stats(s_ref[...], q_ref[...], float(_T))
    lo = m - _THR * sd
    hi = m + _THR * sd

    @pl.when(pl.program_id(0) == 0)
    def _():
        ms_ref[...] = jnp.zeros_like(ms_ref)
        mq_ref[...] = jnp.zeros_like(mq_ref)
        mc_ref[...] = jnp.zeros_like(mc_ref)

    xb = x_ref[...]
    msk = (xb >= lo) & (xb <= hi)
    xm = jnp.where(msk, xb, 0.0)
    x3 = xm.reshape(_BR // 8, 8, _H)
    c3 = msk.astype(jnp.float32).reshape(_BR // 8, 8, _H)
    ms_ref[...] += jnp.sum(x3, axis=0)
    mq_ref[...] += jnp.sum(x3 * x3, axis=0)
    mc_ref[...] += jnp.sum(c3, axis=0)


def _p3_body(x_ref, ms_ref, mq_ref, mc_ref, cs_ref, cq_ref):
    lo, hi = _masked_bounds(ms_ref[...], mq_ref[...], mc_ref[...])

    @pl.when(pl.program_id(0) == 0)
    def _():
        cs_ref[...] = jnp.zeros_like(cs_ref)
        cq_ref[...] = jnp.zeros_like(cq_ref)

    xc = jnp.clip(x_ref[...], lo, hi)
    x3 = xc.reshape(_BR // 8, 8, _H)
    cs_ref[...] += jnp.sum(x3, axis=0)
    cq_ref[...] += jnp.sum(x3 * x3, axis=0)


def _p4_body(x_ref, ms_ref, mq_ref, mc_ref, cs_ref, cq_ref, o_ref):
    lo, hi = _masked_bounds(ms_ref[...], mq_ref[...], mc_ref[...])
    m2, sd2 = _final_stats(cs_ref[...], cq_ref[...], float(_T))
    r2 = 1.0 / sd2
    xc = jnp.clip(x_ref[...], lo, hi)
    o_ref[...] = jnp.clip((xc - m2) * r2, -_CLIP, _CLIP)


_S8 = jax.ShapeDtypeStruct((8, _H), jnp.float32)
_stat_spec = pl.BlockSpec((8, _H), lambda i: (0, 0))
_x_spec = pl.BlockSpec((_BR, _H), lambda i: (i, 0))
_params = pltpu.CompilerParams(dimension_semantics=("arbitrary",))


def _run_p1(x):
    return pl.pallas_call(
        _p1_body,
        grid=(_NBLK,),
        in_specs=[_x_spec],
        out_specs=(_stat_spec, _stat_spec),
        out_shape=(_S8, _S8),
        compiler_params=_params,
    )(x)


def _run_p2(x, s8, q8):
    return pl.pallas_call(
        _p2_body,
        grid=(_NBLK,),
        in_specs=[_x_spec, _stat_spec, _stat_spec],
        out_specs=(_stat_spec, _stat_spec, _stat_spec),
        out_shape=(_S8, _S8, _S8),
        compiler_params=_params,
    )(x, s8, q8)


def _run_p3(x, ms8, mq8, mc8):
    return pl.pallas_call(
        _p3_body,
        grid=(_NBLK,),
        in_specs=[_x_spec, _stat_spec, _stat_spec, _stat_spec],
        out_specs=(_stat_spec, _stat_spec),
        out_shape=(_S8, _S8),
        compiler_params=_params,
    )(x, ms8, mq8, mc8)


def _run_p4(x, ms8, mq8, mc8, cs8, cq8):
    return pl.pallas_call(
        _p4_body,
        grid=(_NBLK,),
        in_specs=[_x_spec] + [_stat_spec] * 5,
        out_specs=_x_spec,
        out_shape=jax.ShapeDtypeStruct((_T, _H), jnp.float32),
        compiler_params=_params,
    )(x, ms8, mq8, mc8, cs8, cq8)


_NC = 2
_NS = 16
_NW = _NC * _NS
_YB = 2000
_NYB = _T // _YB
_BPW = -(-_NYB // _NW)


def _build_labels_sc():
    return functools.partial(
        pl.kernel,
        mesh=plsc.VectorSubcoreMesh(core_axis_name="c", subcore_axis_name="s"),
        out_type=jax.ShapeDtypeStruct((_T,), jnp.int32),
        scratch_types=[
            pltpu.VMEM((16 * (_NCLS - 1),), jnp.int32),
            pltpu.VMEM((16 * (_NCLS - 1),), jnp.float32),
            pltpu.VMEM((_YB,), jnp.float32),
            pltpu.VMEM((_YB,), jnp.int32),
            pltpu.SemaphoreType.DMA,
        ],
    )(_labels_sc_body)


def _labels_sc_body(y_hbm, idx_hbm, out_hbm, idx_v, b_v, y_v, o_v, sem):
    wid = lax.axis_index("s") * _NC + lax.axis_index("c")
    pltpu.sync_copy(idx_hbm, idx_v)
    pltpu.async_copy(y_hbm.at[idx_v], b_v, sem).wait()
    bvecs = [b_v[pl.ds(16 * j, 16)] for j in range(_NCLS - 1)]

    for t in range(_BPW):
        blk = wid + t * _NW

        @pl.when(blk < _NYB)
        def _():
            base = blk * _YB
            pltpu.sync_copy(y_hbm.at[pl.ds(base, _YB)], y_v)

            def body(i, carry):
                v = y_v[pl.ds(i * 16, 16)]
                acc = jnp.zeros((16,), jnp.int32)
                for bj in bvecs:
                    acc = acc + jnp.where(v > bj, 1, 0)
                o_v[pl.ds(i * 16, 16)] = acc
                return carry

            lax.fori_loop(0, _YB // 16, body, 0)
            pltpu.sync_copy(o_v, out_hbm.at[pl.ds(base, _YB)])


def kernel(x, y):
    s8, q8 = _run_p1(x)
    ms8, mq8, mc8 = _run_p2(x, s8, q8)
    cs8, cq8 = _run_p3(x, ms8, mq8, mc8)
    x_proc = _run_p4(x, ms8, mq8, mc8, cs8, cq8)

    bidx = jax.random.randint(jax.random.key(42), (_NCLS - 1,), 0, _T)
    idx_rep = jnp.repeat(bidx.astype(jnp.int32), 16)
    labels = _build_labels_sc()(y, idx_rep)
    return x_proc, labels


# speedup vs baseline: 2.0827x; 1.2573x over previous
import functools

import jax
import jax.numpy as jnp
from jax import lax
from jax.experimental import pallas as pl
from jax.experimental.pallas import tpu as pltpu
from jax.experimental.pallas import tpu_sc as plsc

_T = 500000
_H = 128
_NCLS = 10
_THR = 4.0
_CLIP = 100.0

_BR = 10000
_NBLK = _T // _BR


def _final_stats(s8, q8, n):
    s = jnp.sum(s8, axis=0, keepdims=True)
    q = jnp.sum(q8, axis=0, keepdims=True)
    m = s / n
    v = jnp.maximum((q - n * m * m) / (n - 1.0), 0.0)
    sd = jnp.maximum(jnp.sqrt(v), 1e-6)
    return m, sd


def _masked_bounds(ms8, mq8, mc8):
    s = jnp.sum(ms8, axis=0, keepdims=True)
    q = jnp.sum(mq8, axis=0, keepdims=True)
    c = jnp.sum(mc8, axis=0, keepdims=True)
    m = s / c
    v = jnp.maximum((q - c * m * m) / (c - 1.0), 0.0)
    sd = jnp.maximum(jnp.sqrt(v), 1e-6)
    return m - _THR * sd, m + _THR * sd


def _p1_body(x_ref, s_ref, q_ref):
    @pl.when(pl.program_id(0) == 0)
    def _():
        s_ref[...] = jnp.zeros_like(s_ref)
        q_ref[...] = jnp.zeros_like(q_ref)

    x3 = x_ref[...].reshape(_BR // 8, 8, _H)
    s_ref[...] += jnp.sum(x3, axis=0)
    q_ref[...] += jnp.sum(x3 * x3, axis=0)


def _p2_body(x_ref, s_ref, q_ref, ms_ref, mq_ref, mc_ref):
    m, sd = _final_stats(s_ref[...], q_ref[...], float(_T))
    lo = m - _THR * sd
    hi = m + _THR * sd

    @pl.when(pl.program_id(0) == 0)
    def _():
        ms_ref[...] = jnp.zeros_like(ms_ref)
        mq_ref[...] = jnp.zeros_like(mq_ref)
        mc_ref[...] = jnp.zeros_like(mc_ref)

    xb = x_ref[...]
    msk = (xb >= lo) & (xb <= hi)
    xm = jnp.where(msk, xb, 0.0)
    x3 = xm.reshape(_BR // 8, 8, _H)
    c3 = msk.astype(jnp.float32).reshape(_BR // 8, 8, _H)
    ms_ref[...] += jnp.sum(x3, axis=0)
    mq_ref[...] += jnp.sum(x3 * x3, axis=0)
    mc_ref[...] += jnp.sum(c3, axis=0)


def _p3_body(x_ref, ms_ref, mq_ref, mc_ref, cs_ref, cq_ref):
    lo, hi = _masked_bounds(ms_ref[...], mq_ref[...], mc_ref[...])

    @pl.when(pl.program_id(0) == 0)
    def _():
        cs_ref[...] = jnp.zeros_like(cs_ref)
        cq_ref[...] = jnp.zeros_like(cq_ref)

    xc = jnp.clip(x_ref[...], lo, hi)
    x3 = xc.reshape(_BR // 8, 8, _H)
    cs_ref[...] += jnp.sum(x3, axis=0)
    cq_ref[...] += jnp.sum(x3 * x3, axis=0)


def _p4_body(x_ref, ms_ref, mq_ref, mc_ref, cs_ref, cq_ref, o_ref):
    lo, hi = _masked_bounds(ms_ref[...], mq_ref[...], mc_ref[...])
    m2, sd2 = _final_stats(cs_ref[...], cq_ref[...], float(_T))
    r2 = 1.0 / sd2
    xc = jnp.clip(x_ref[...], lo, hi)
    o_ref[...] = jnp.clip((xc - m2) * r2, -_CLIP, _CLIP)


_S8 = jax.ShapeDtypeStruct((8, _H), jnp.float32)
_stat_spec = pl.BlockSpec((8, _H), lambda i: (0, 0))
_x_spec = pl.BlockSpec((_BR, _H), lambda i: (i, 0))
_params = pltpu.CompilerParams(dimension_semantics=("arbitrary",))


def _run_p1(x):
    return pl.pallas_call(
        _p1_body,
        grid=(_NBLK,),
        in_specs=[_x_spec],
        out_specs=(_stat_spec, _stat_spec),
        out_shape=(_S8, _S8),
        compiler_params=_params,
    )(x)


def _run_p2(x, s8, q8):
    return pl.pallas_call(
        _p2_body,
        grid=(_NBLK,),
        in_specs=[_x_spec, _stat_spec, _stat_spec],
        out_specs=(_stat_spec, _stat_spec, _stat_spec),
        out_shape=(_S8, _S8, _S8),
        compiler_params=_params,
    )(x, s8, q8)


def _run_p3(x, ms8, mq8, mc8):
    return pl.pallas_call(
        _p3_body,
        grid=(_NBLK,),
        in_specs=[_x_spec, _stat_spec, _stat_spec, _stat_spec],
        out_specs=(_stat_spec, _stat_spec),
        out_shape=(_S8, _S8),
        compiler_params=_params,
    )(x, ms8, mq8, mc8)


def _run_p4(x, ms8, mq8, mc8, cs8, cq8):
    return pl.pallas_call(
        _p4_body,
        grid=(_NBLK,),
        in_specs=[_x_spec] + [_stat_spec] * 5,
        out_specs=_x_spec,
        out_shape=jax.ShapeDtypeStruct((_T, _H), jnp.float32),
        compiler_params=_params,
    )(x, ms8, mq8, mc8, cs8, cq8)


_NC = 2
_NS = 16
_NW = _NC * _NS
_YB = 2000
_NYB = _T // _YB
_BPW = -(-_NYB // _NW)


def _build_labels_sc():
    return functools.partial(
        pl.kernel,
        mesh=plsc.VectorSubcoreMesh(core_axis_name="c", subcore_axis_name="s"),
        out_type=jax.ShapeDtypeStruct((_T,), jnp.int32),
        scratch_types=[
            pltpu.VMEM((16 * (_NCLS - 1),), jnp.int32),
            pltpu.VMEM((16 * (_NCLS - 1),), jnp.float32),
            pltpu.VMEM((_YB,), jnp.float32),
            pltpu.VMEM((_YB,), jnp.int32),
            pltpu.SemaphoreType.DMA,
        ],
    )(_labels_sc_body)


def _labels_sc_body(y_hbm, idx_hbm, out_hbm, idx_v, b_v, y_v, o_v, sem):
    wid = lax.axis_index("s") * _NC + lax.axis_index("c")
    pltpu.sync_copy(idx_hbm, idx_v)
    pltpu.async_copy(y_hbm.at[idx_v], b_v, sem).wait()
    bvecs = [b_v[pl.ds(16 * j, 16)] for j in range(_NCLS - 1)]

    for t in range(_BPW):
        blk = wid + t * _NW

        @pl.when(blk < _NYB)
        def _():
            base = blk * _YB
            pltpu.sync_copy(y_hbm.at[pl.ds(base, _YB)], y_v)

            def body(i, carry):
                v = y_v[pl.ds(i * 16, 16)]
                acc = jnp.zeros((16,), jnp.int32)
                for bj in bvecs:
                    acc = acc + jnp.where(v > bj, 1, 0)
                o_v[pl.ds(i * 16, 16)] = acc
                return carry

            lax.fori_loop(0, _YB // 16, body, 0)
            pltpu.sync_copy(o_v, out_hbm.at[pl.ds(base, _YB)])


def kernel(x, y):
    s8, q8 = _run_p1(x)
    ms8, mq8, mc8 = _run_p2(x, s8, q8)
    cs8, cq8 = _run_p3(x, ms8, mq8, mc8)
    x_proc = _run_p4(x, ms8, mq8, mc8, cs8, cq8)

    bidx = jax.random.randint(jax.random.key(42), (_NCLS - 1,), 0, _T)
    idx_rep = jnp.repeat(bidx.astype(jnp.int32), 16)
    labels = _build_labels_sc()(y, idx_rep)
    return x_proc, labels


# speedup vs baseline: 2.2338x; 1.0725x over previous
import functools

import jax
import jax.numpy as jnp
from jax import lax
from jax.experimental import pallas as pl
from jax.experimental.pallas import tpu as pltpu
from jax.experimental.pallas import tpu_sc as plsc

_T = 500000
_H = 128
_NCLS = 10
_THR = 4.0
_CLIP = 100.0

_BR = 20000
_NBLK = _T // _BR


def _final_stats(s8, q8, n):
    s = jnp.sum(s8, axis=0, keepdims=True)
    q = jnp.sum(q8, axis=0, keepdims=True)
    m = s / n
    v = jnp.maximum((q - n * m * m) / (n - 1.0), 0.0)
    sd = jnp.maximum(jnp.sqrt(v), 1e-6)
    return m, sd


def _masked_bounds(ms8, mq8, mc8):
    s = jnp.sum(ms8, axis=0, keepdims=True)
    q = jnp.sum(mq8, axis=0, keepdims=True)
    c = jnp.sum(mc8, axis=0, keepdims=True)
    m = s / c
    v = jnp.maximum((q - c * m * m) / (c - 1.0), 0.0)
    sd = jnp.maximum(jnp.sqrt(v), 1e-6)
    return m - _THR * sd, m + _THR * sd


def _p1_body(x_ref, s_ref, q_ref):
    @pl.when(pl.program_id(0) == 0)
    def _():
        s_ref[...] = jnp.zeros_like(s_ref)
        q_ref[...] = jnp.zeros_like(q_ref)

    x3 = x_ref[...].reshape(_BR // 8, 8, _H)
    s_ref[...] += jnp.sum(x3, axis=0)
    q_ref[...] += jnp.sum(x3 * x3, axis=0)


def _p2_body(x_ref, s_ref, q_ref, ms_ref, mq_ref, mc_ref):
    m, sd = _final_stats(s_ref[...], q_ref[...], float(_T))
    lo = m - _THR * sd
    hi = m + _THR * sd

    @pl.when(pl.program_id(0) == 0)
    def _():
        ms_ref[...] = jnp.zeros_like(ms_ref)
        mq_ref[...] = jnp.zeros_like(mq_ref)
        mc_ref[...] = jnp.zeros_like(mc_ref)

    xb = x_ref[...]
    msk = (xb >= lo) & (xb <= hi)
    xm = jnp.where(msk, xb, 0.0)
    x3 = xm.reshape(_BR // 8, 8, _H)
    c3 = msk.astype(jnp.float32).reshape(_BR // 8, 8, _H)
    ms_ref[...] += jnp.sum(x3, axis=0)
    mq_ref[...] += jnp.sum(x3 * x3, axis=0)
    mc_ref[...] += jnp.sum(c3, axis=0)


def _p3_body(x_ref, ms_ref, mq_ref, mc_ref, cs_ref, cq_ref):
    lo, hi = _masked_bounds(ms_ref[...], mq_ref[...], mc_ref[...])

    @pl.when(pl.program_id(0) == 0)
    def _():
        cs_ref[...] = jnp.zeros_like(cs_ref)
        cq_ref[...] = jnp.zeros_like(cq_ref)

    xc = jnp.clip(x_ref[...], lo, hi)
    x3 = xc.reshape(_BR // 8, 8, _H)
    cs_ref[...] += jnp.sum(x3, axis=0)
    cq_ref[...] += jnp.sum(x3 * x3, axis=0)


def _p4_body(x_ref, ms_ref, mq_ref, mc_ref, cs_ref, cq_ref, o_ref):
    lo, hi = _masked_bounds(ms_ref[...], mq_ref[...], mc_ref[...])
    m2, sd2 = _final_stats(cs_ref[...], cq_ref[...], float(_T))
    r2 = 1.0 / sd2
    xc = jnp.clip(x_ref[...], lo, hi)
    o_ref[...] = jnp.clip((xc - m2) * r2, -_CLIP, _CLIP)


_S8 = jax.ShapeDtypeStruct((8, _H), jnp.float32)
_stat_spec = pl.BlockSpec((8, _H), lambda i: (0, 0))
_x_spec = pl.BlockSpec((_BR, _H), lambda i: (i, 0))
_params = pltpu.CompilerParams(dimension_semantics=("arbitrary",))


def _run_p1(x):
    return pl.pallas_call(
        _p1_body,
        grid=(_NBLK,),
        in_specs=[_x_spec],
        out_specs=(_stat_spec, _stat_spec),
        out_shape=(_S8, _S8),
        compiler_params=_params,
    )(x)


def _run_p2(x, s8, q8):
    return pl.pallas_call(
        _p2_body,
        grid=(_NBLK,),
        in_specs=[_x_spec, _stat_spec, _stat_spec],
        out_specs=(_stat_spec, _stat_spec, _stat_spec),
        out_shape=(_S8, _S8, _S8),
        compiler_params=_params,
    )(x, s8, q8)


def _run_p3(x, ms8, mq8, mc8):
    return pl.pallas_call(
        _p3_body,
        grid=(_NBLK,),
        in_specs=[_x_spec, _stat_spec, _stat_spec, _stat_spec],
        out_specs=(_stat_spec, _stat_spec),
        out_shape=(_S8, _S8),
        compiler_params=_params,
    )(x, ms8, mq8, mc8)


def _run_p4(x, ms8, mq8, mc8, cs8, cq8):
    return pl.pallas_call(
        _p4_body,
        grid=(_NBLK,),
        in_specs=[_x_spec] + [_stat_spec] * 5,
        out_specs=_x_spec,
        out_shape=jax.ShapeDtypeStruct((_T, _H), jnp.float32),
        compiler_params=_params,
    )(x, ms8, mq8, mc8, cs8, cq8)


_NC = 2
_NS = 16
_NW = _NC * _NS
_YB = 2000
_NYB = _T // _YB
_BPW = -(-_NYB // _NW)


def _build_labels_sc():
    return functools.partial(
        pl.kernel,
        mesh=plsc.VectorSubcoreMesh(core_axis_name="c", subcore_axis_name="s"),
        out_type=jax.ShapeDtypeStruct((_T,), jnp.int32),
        scratch_types=[
            pltpu.VMEM((16 * (_NCLS - 1),), jnp.int32),
            pltpu.VMEM((16 * (_NCLS - 1),), jnp.float32),
            pltpu.VMEM((_YB,), jnp.float32),
            pltpu.VMEM((_YB,), jnp.int32),
            pltpu.SemaphoreType.DMA,
        ],
    )(_labels_sc_body)


def _labels_sc_body(y_hbm, idx_hbm, out_hbm, idx_v, b_v, y_v, o_v, sem):
    wid = lax.axis_index("s") * _NC + lax.axis_index("c")
    pltpu.sync_copy(idx_hbm, idx_v)
    pltpu.async_copy(y_hbm.at[idx_v], b_v, sem).wait()
    bvecs = [b_v[pl.ds(16 * j, 16)] for j in range(_NCLS - 1)]

    for t in range(_BPW):
        blk = wid + t * _NW

        @pl.when(blk < _NYB)
        def _():
            base = blk * _YB
            pltpu.sync_copy(y_hbm.at[pl.ds(base, _YB)], y_v)

            def body(i, carry):
                v = y_v[pl.ds(i * 16, 16)]
                acc = jnp.zeros((16,), jnp.int32)
                for bj in bvecs:
                    acc = acc + jnp.where(v > bj, 1, 0)
                o_v[pl.ds(i * 16, 16)] = acc
                return carry

            lax.fori_loop(0, _YB // 16, body, 0)
            pltpu.sync_copy(o_v, out_hbm.at[pl.ds(base, _YB)])


def kernel(x, y):
    s8, q8 = _run_p1(x)
    ms8, mq8, mc8 = _run_p2(x, s8, q8)
    cs8, cq8 = _run_p3(x, ms8, mq8, mc8)
    x_proc = _run_p4(x, ms8, mq8, mc8, cs8, cq8)

    bidx = jax.random.randint(jax.random.key(42), (_NCLS - 1,), 0, _T)
    idx_rep = jnp.repeat(bidx.astype(jnp.int32), 16)
    labels = _build_labels_sc()(y, idx_rep)
    return x_proc, labels


# speedup vs baseline: 2.2573x; 1.0105x over previous
import functools

import jax
import jax.numpy as jnp
from jax import lax
from jax.experimental import pallas as pl
from jax.experimental.pallas import tpu as pltpu
from jax.experimental.pallas import tpu_sc as plsc

_T = 500000
_H = 128
_NCLS = 10
_THR = 4.0
_CLIP = 100.0

_BR = 25000
_NBLK = _T // _BR


def _final_stats(s8, q8, n):
    s = jnp.sum(s8, axis=0, keepdims=True)
    q = jnp.sum(q8, axis=0, keepdims=True)
    m = s / n
    v = jnp.maximum((q - n * m * m) / (n - 1.0), 0.0)
    sd = jnp.maximum(jnp.sqrt(v), 1e-6)
    return m, sd


def _masked_bounds(ms8, mq8, mc8):
    s = jnp.sum(ms8, axis=0, keepdims=True)
    q = jnp.sum(mq8, axis=0, keepdims=True)
    c = jnp.sum(mc8, axis=0, keepdims=True)
    m = s / c
    v = jnp.maximum((q - c * m * m) / (c - 1.0), 0.0)
    sd = jnp.maximum(jnp.sqrt(v), 1e-6)
    return m - _THR * sd, m + _THR * sd


def _p1_body(x_ref, s_ref, q_ref):
    @pl.when(pl.program_id(0) == 0)
    def _():
        s_ref[...] = jnp.zeros_like(s_ref)
        q_ref[...] = jnp.zeros_like(q_ref)

    x3 = x_ref[...].reshape(_BR // 8, 8, _H)
    s_ref[...] += jnp.sum(x3, axis=0)
    q_ref[...] += jnp.sum(x3 * x3, axis=0)


def _p2_body(x_ref, s_ref, q_ref, ms_ref, mq_ref, mc_ref):
    m, sd = _final_stats(s_ref[...], q_ref[...], float(_T))
    lo = m - _THR * sd
    hi = m + _THR * sd

    @pl.when(pl.program_id(0) == 0)
    def _():
        ms_ref[...] = jnp.zeros_like(ms_ref)
        mq_ref[...] = jnp.zeros_like(mq_ref)
        mc_ref[...] = jnp.zeros_like(mc_ref)

    xb = x_ref[...]
    msk = (xb >= lo) & (xb <= hi)
    xm = jnp.where(msk, xb, 0.0)
    x3 = xm.reshape(_BR // 8, 8, _H)
    c3 = msk.astype(jnp.float32).reshape(_BR // 8, 8, _H)
    ms_ref[...] += jnp.sum(x3, axis=0)
    mq_ref[...] += jnp.sum(x3 * x3, axis=0)
    mc_ref[...] += jnp.sum(c3, axis=0)


def _p3_body(x_ref, ms_ref, mq_ref, mc_ref, cs_ref, cq_ref):
    lo, hi = _masked_bounds(ms_ref[...], mq_ref[...], mc_ref[...])

    @pl.when(pl.program_id(0) == 0)
    def _():
        cs_ref[...] = jnp.zeros_like(cs_ref)
        cq_ref[...] = jnp.zeros_like(cq_ref)

    xc = jnp.clip(x_ref[...], lo, hi)
    x3 = xc.reshape(_BR // 8, 8, _H)
    cs_ref[...] += jnp.sum(x3, axis=0)
    cq_ref[...] += jnp.sum(x3 * x3, axis=0)


def _p4_body(x_ref, ms_ref, mq_ref, mc_ref, cs_ref, cq_ref, o_ref):
    lo, hi = _masked_bounds(ms_ref[...], mq_ref[...], mc_ref[...])
    m2, sd2 = _final_stats(cs_ref[...], cq_ref[...], float(_T))
    r2 = 1.0 / sd2
    xc = jnp.clip(x_ref[...], lo, hi)
    o_ref[...] = jnp.clip((xc - m2) * r2, -_CLIP, _CLIP)


_S8 = jax.ShapeDtypeStruct((8, _H), jnp.float32)
_stat_spec = pl.BlockSpec((8, _H), lambda i: (0, 0))
_x_spec = pl.BlockSpec((_BR, _H), lambda i: (i, 0))
_params = pltpu.CompilerParams(dimension_semantics=("arbitrary",))


def _run_p1(x):
    return pl.pallas_call(
        _p1_body,
        grid=(_NBLK,),
        in_specs=[_x_spec],
        out_specs=(_stat_spec, _stat_spec),
        out_shape=(_S8, _S8),
        compiler_params=_params,
    )(x)


def _run_p2(x, s8, q8):
    return pl.pallas_call(
        _p2_body,
        grid=(_NBLK,),
        in_specs=[_x_spec, _stat_spec, _stat_spec],
        out_specs=(_stat_spec, _stat_spec, _stat_spec),
        out_shape=(_S8, _S8, _S8),
        compiler_params=_params,
    )(x, s8, q8)


def _run_p3(x, ms8, mq8, mc8):
    return pl.pallas_call(
        _p3_body,
        grid=(_NBLK,),
        in_specs=[_x_spec, _stat_spec, _stat_spec, _stat_spec],
        out_specs=(_stat_spec, _stat_spec),
        out_shape=(_S8, _S8),
        compiler_params=_params,
    )(x, ms8, mq8, mc8)


def _run_p4(x, ms8, mq8, mc8, cs8, cq8):
    return pl.pallas_call(
        _p4_body,
        grid=(_NBLK,),
        in_specs=[_x_spec] + [_stat_spec] * 5,
        out_specs=_x_spec,
        out_shape=jax.ShapeDtypeStruct((_T, _H), jnp.float32),
        compiler_params=_params,
    )(x, ms8, mq8, mc8, cs8, cq8)


_NC = 2
_NS = 16
_NW = _NC * _NS
_YB = 2000
_NYB = _T // _YB
_BPW = -(-_NYB // _NW)


def _build_labels_sc():
    return functools.partial(
        pl.kernel,
        mesh=plsc.VectorSubcoreMesh(core_axis_name="c", subcore_axis_name="s"),
        out_type=jax.ShapeDtypeStruct((_T,), jnp.int32),
        scratch_types=[
            pltpu.VMEM((16 * (_NCLS - 1),), jnp.int32),
            pltpu.VMEM((16 * (_NCLS - 1),), jnp.float32),
            pltpu.VMEM((_YB,), jnp.float32),
            pltpu.VMEM((_YB,), jnp.int32),
            pltpu.SemaphoreType.DMA,
        ],
    )(_labels_sc_body)


def _labels_sc_body(y_hbm, idx_hbm, out_hbm, idx_v, b_v, y_v, o_v, sem):
    wid = lax.axis_index("s") * _NC + lax.axis_index("c")
    pltpu.sync_copy(idx_hbm, idx_v)
    pltpu.async_copy(y_hbm.at[idx_v], b_v, sem).wait()
    bvecs = [b_v[pl.ds(16 * j, 16)] for j in range(_NCLS - 1)]

    for t in range(_BPW):
        blk = wid + t * _NW

        @pl.when(blk < _NYB)
        def _():
            base = blk * _YB
            pltpu.sync_copy(y_hbm.at[pl.ds(base, _YB)], y_v)

            def body(i, carry):
                v = y_v[pl.ds(i * 16, 16)]
                acc = jnp.zeros((16,), jnp.int32)
                for bj in bvecs:
                    acc = acc + jnp.where(v > bj, 1, 0)
                o_v[pl.ds(i * 16, 16)] = acc
                return carry

            lax.fori_loop(0, _YB // 16, body, 0)
            pltpu.sync_copy(o_v, out_hbm.at[pl.ds(base, _YB)])


def kernel(x, y):
    s8, q8 = _run_p1(x)
    ms8, mq8, mc8 = _run_p2(x, s8, q8)
    cs8, cq8 = _run_p3(x, ms8, mq8, mc8)
    x_proc = _run_p4(x, ms8, mq8, mc8, cs8, cq8)

    bidx = jax.random.randint(jax.random.key(42), (_NCLS - 1,), 0, _T)
    idx_rep = jnp.repeat(bidx.astype(jnp.int32), 16)
    labels = _build_labels_sc()(y, idx_rep)
    return x_proc, labels
